# baseline (speedup 1.0000x reference)
import dataclasses
import functools

import jax
import jax.numpy as jnp
from jax import lax
from jax.experimental import pallas as pl
from jax.experimental.pallas import tpu as pltpu
from jax.experimental.pallas import tpu_sc as plsc

_NUM_CORES = 2
_NUM_SUBCORES = 16
_NUM_WORKERS = _NUM_CORES * _NUM_SUBCORES
_LANES = 16

_SCALE = 1.4142135623730951


def _compiler_params():
    cp = pltpu.CompilerParams(use_tc_tiling_on_sc=False)
    if "needs_layout_passes" in pltpu.CompilerParams.__dataclass_fields__:
        cp = dataclasses.replace(cp, needs_layout_passes=False)
    return cp


@functools.cache
def _build(N, T, D):
    BN = 128
    per_w = N // _NUM_WORKERS
    nb = per_w // BN
    assert per_w % BN == 0

    mesh = plsc.VectorSubcoreMesh(core_axis_name="c", subcore_axis_name="s")

    @functools.partial(
        pl.kernel,
        mesh=mesh,
        compiler_params=_compiler_params(),
        out_type=jax.ShapeDtypeStruct((T, D, N), jnp.float32),
        scratch_types=[
            pltpu.VMEM((BN,), jnp.int32),
            pltpu.VMEM((BN, D), jnp.float32),
            pltpu.VMEM((BN, D), jnp.float32),
            pltpu.VMEM((D, BN), jnp.float32),
            pltpu.SemaphoreType.DMA,
            pltpu.SemaphoreType.DMA,
        ],
    )
    def fused(xt_hbm, wa_hbm, wb_hbm, out_hbm, idx_v, a_v, b_v, o_v, sem_a, sem_b):
        wid = lax.axis_index("s") * _NUM_CORES + lax.axis_index("c")
        n0w = wid * per_w
        scale = jnp.float32(_SCALE)
        lane = lax.iota(jnp.int32, 16)

        @pl.loop(0, T * nb)
        def _(bi):
            t = bi // nb
            n0 = n0w + (bi % nb) * BN
            pltpu.sync_copy(xt_hbm.at[t, pl.ds(n0, BN)], idx_v)
            cp_a = pltpu.async_copy(wa_hbm.at[idx_v], a_v, sem_a)
            cp_b = pltpu.async_copy(wb_hbm.at[idx_v], b_v, sem_b)
            cp_a.wait()
            cp_b.wait()

            @pl.loop(0, D)
            def _(d):
                dv = jnp.full((16,), d, dtype=jnp.int32)
                for g in range(BN // _LANES):
                    rows = lane + (g * _LANES)
                    av = plsc.load_gather(a_v, [rows, dv])
                    bv = plsc.load_gather(b_v, [rows, dv])
                    o_v[d, pl.ds(g * _LANES, _LANES)] = av + scale * bv

            pltpu.sync_copy(o_v, out_hbm.at[t, :, pl.ds(n0, BN)])

    return fused


@jax.jit
def kernel(x, Wa, Wb):
    N, T = x.shape
    D = Wa.shape[1]
    out_t = _build(N, T, D)(x.T.astype(jnp.int32), Wa, Wb)
    return out_t.transpose(2, 0, 1)


# speedup vs baseline: 1.8619x; 1.8619x over previous
import dataclasses
import functools

import jax
import jax.numpy as jnp
from jax import lax
from jax.experimental import pallas as pl
from jax.experimental.pallas import tpu as pltpu
from jax.experimental.pallas import tpu_sc as plsc

_NUM_CORES = 2
_NUM_SUBCORES = 16
_NUM_WORKERS = _NUM_CORES * _NUM_SUBCORES
_LANES = 16

_SCALE = 1.4142135623730951


def _compiler_params():
    cp = pltpu.CompilerParams(use_tc_tiling_on_sc=False)
    if "needs_layout_passes" in pltpu.CompilerParams.__dataclass_fields__:
        cp = dataclasses.replace(cp, needs_layout_passes=False)
    return cp


@functools.cache
def _build(N, T, D):
    G = 128
    SUBS = N // G
    JW = SUBS // _NUM_WORKERS
    JB = 2
    NB = JB * G
    n_blocks = T * (JW // JB)
    OP = NB + 1

    mesh = plsc.VectorSubcoreMesh(core_axis_name="c", subcore_axis_name="s")

    @functools.partial(
        pl.kernel,
        mesh=mesh,
        compiler_params=_compiler_params(),
        out_type=jax.ShapeDtypeStruct((T, D, N), jnp.float32),
        scratch_types=[
            pltpu.VMEM((T, JW, G), jnp.int32),
            pltpu.VMEM((2, JB, G, D), jnp.float32),
            pltpu.VMEM((2, JB, G, D), jnp.float32),
            pltpu.VMEM((2, D, OP), jnp.float32),
            pltpu.SemaphoreType.DMA,
            pltpu.SemaphoreType.DMA,
            pltpu.SemaphoreType.DMA,
            pltpu.SemaphoreType.DMA,
            pltpu.SemaphoreType.DMA,
        ],
    )
    def fused(xt_hbm, wa_hbm, wb_hbm, out_hbm, idx_v, a_v, b_v, o_v,
              sem_i, semg0, semg1, semo0, semo1):
        wid = lax.axis_index("s") * _NUM_CORES + lax.axis_index("c")
        scale = jnp.float32(_SCALE)
        lane = lax.iota(jnp.int32, 16)
        lanes = [lane + (h * _LANES) for h in range(D // _LANES)]
        semg = [semg0, semg1]
        semo = [semo0, semo1]

        pltpu.async_copy(xt_hbm.at[:, pl.ds(wid * JW, JW)], idx_v, sem_i).wait()

        def issue_gathers(bi, s):
            t = bi // (JW // JB)
            jb0 = (bi % (JW // JB)) * JB
            for jj in range(JB):
                pltpu.async_copy(
                    wa_hbm.at[idx_v.at[t, jb0 + jj]], a_v.at[s, jj], semg[s]
                )
                pltpu.async_copy(
                    wb_hbm.at[idx_v.at[t, jb0 + jj]], b_v.at[s, jj], semg[s]
                )

        def drain_gathers(bi, s):
            t = bi // (JW // JB)
            jb0 = (bi % (JW // JB)) * JB
            for jj in range(JB):
                pltpu.make_async_copy(
                    wa_hbm.at[idx_v.at[t, jb0 + jj]], a_v.at[s, jj], semg[s]
                ).wait()
                pltpu.make_async_copy(
                    wb_hbm.at[idx_v.at[t, jb0 + jj]], b_v.at[s, jj], semg[s]
                ).wait()

        def out_copy(bi, s):
            t = bi // (JW // JB)
            col0 = (wid * JW + (bi % (JW // JB)) * JB) * G
            return pltpu.make_async_copy(
                o_v.at[s, :, pl.ds(0, NB)],
                out_hbm.at[t, :, pl.ds(col0, NB)],
                semo[s],
            )

        def run_block(bi, s, issue_next):
            if issue_next:
                issue_gathers(bi + 1, 1 - s)
            drain_gathers(bi, s)

            def drain_prev():
                out_copy(bi - 2, s).wait()

            if isinstance(bi, int):
                if bi >= 2:
                    drain_prev()
            else:
                pl.when(bi >= 2)(drain_prev)

            for jj in range(JB):
                @pl.loop(0, G, step=4)
                def _(r0):
                    for dr in range(4):
                        r = r0 + dr
                        nv = jnp.full((16,), jj * G, jnp.int32) + r
                        for h in range(D // _LANES):
                            sl = pl.ds(h * _LANES, _LANES)
                            av = a_v[s, jj, r, sl]
                            bv = b_v[s, jj, r, sl]
                            plsc.store_scatter(
                                o_v.at[s], [lanes[h], nv], av + scale * bv
                            )

            out_copy(bi, s).start()

        issue_gathers(0, 0)

        @pl.loop(0, (n_blocks - 2) // 2)
        def _(k):
            bi = k * 2
            run_block(bi, 0, True)
            run_block(bi + 1, 1, True)

        run_block(n_blocks - 2, 0, True)
        run_block(n_blocks - 1, 1, False)
        out_copy(n_blocks - 2, (n_blocks - 2) % 2).wait()
        out_copy(n_blocks - 1, (n_blocks - 1) % 2).wait()

    return fused


@jax.jit
def kernel(x, Wa, Wb):
    N, T = x.shape
    D = Wa.shape[1]
    xt3 = x.T.astype(jnp.int32).reshape(T, N // 128, 128)
    out_t = _build(N, T, D)(xt3, Wa, Wb)
    return out_t.transpose(2, 0, 1)


# speedup vs baseline: 1.9940x; 1.0709x over previous
import dataclasses
import functools

import jax
import jax.numpy as jnp
from jax import lax
from jax.experimental import pallas as pl
from jax.experimental.pallas import tpu as pltpu
from jax.experimental.pallas import tpu_sc as plsc

_NUM_CORES = 2
_NUM_SUBCORES = 16
_NUM_WORKERS = _NUM_CORES * _NUM_SUBCORES
_LANES = 16

_SCALE = 1.4142135623730951


def _compiler_params():
    cp = pltpu.CompilerParams(use_tc_tiling_on_sc=False)
    if "needs_layout_passes" in pltpu.CompilerParams.__dataclass_fields__:
        cp = dataclasses.replace(cp, needs_layout_passes=False)
    return cp


@functools.cache
def _build(N, T, D):
    G = 64
    W128 = 128
    SUBS = N // G
    JW = SUBS // _NUM_WORKERS
    JB = 4
    NB = JB * G
    n_blocks = T * (JW // JB)
    OP = NB + 1

    mesh = plsc.VectorSubcoreMesh(core_axis_name="c", subcore_axis_name="s")

    @functools.partial(
        pl.kernel,
        mesh=mesh,
        compiler_params=_compiler_params(),
        out_type=jax.ShapeDtypeStruct((T, D, N), jnp.float32),
        scratch_types=[
            pltpu.VMEM((T, JW, G), jnp.int32),
            pltpu.VMEM((2, JB, G, W128), jnp.float32),
            pltpu.VMEM((2, D, OP), jnp.float32),
            pltpu.SemaphoreType.DMA,
            pltpu.SemaphoreType.DMA,
            pltpu.SemaphoreType.DMA,
            pltpu.SemaphoreType.DMA,
            pltpu.SemaphoreType.DMA,
        ],
    )
    def fused(xt_hbm, w_hbm, out_hbm, idx_v, w_v, o_v,
              sem_i, semg0, semg1, semo0, semo1):
        wid = lax.axis_index("s") * _NUM_CORES + lax.axis_index("c")
        scale = jnp.float32(_SCALE)
        lane = lax.iota(jnp.int32, 16)
        lanes = [lane + (h * _LANES) for h in range(D // _LANES)]
        semg = [semg0, semg1]
        semo = [semo0, semo1]

        pltpu.async_copy(xt_hbm.at[:, pl.ds(wid * JW, JW)], idx_v, sem_i).wait()

        def block_coords(bi):
            t = bi // (JW // JB)
            jb0 = (bi % (JW // JB)) * JB
            return t, jb0

        def issue_gathers(bi, s):
            t, jb0 = block_coords(bi)
            for jj in range(JB):
                pltpu.async_copy(
                    w_hbm.at[idx_v.at[t, jb0 + jj]], w_v.at[s, jj], semg[s]
                )

        def drain_gathers(bi, s):
            t, jb0 = block_coords(bi)
            for jj in range(JB):
                pltpu.make_async_copy(
                    w_hbm.at[idx_v.at[t, jb0 + jj]], w_v.at[s, jj], semg[s]
                ).wait()

        def out_copy(bi, s):
            t, jb0 = block_coords(bi)
            col0 = (wid * JW + jb0) * G
            return pltpu.make_async_copy(
                o_v.at[s, :, pl.ds(0, NB)],
                out_hbm.at[t, :, pl.ds(col0, NB)],
                semo[s],
            )

        def run_block(bi, s, issue_next):
            if issue_next:
                issue_gathers(bi + 1, 1 - s)
            drain_gathers(bi, s)

            def drain_prev():
                out_copy(bi - 2, s).wait()

            if isinstance(bi, int):
                if bi >= 2:
                    drain_prev()
            else:
                pl.when(bi >= 2)(drain_prev)

            for jj in range(JB):
                @pl.loop(0, G, step=4)
                def _(r0):
                    for dr in range(4):
                        r = r0 + dr
                        nv = jnp.full((16,), jj * G, jnp.int32) + r
                        for h in range(D // _LANES):
                            av = w_v[s, jj, r, pl.ds(h * _LANES, _LANES)]
                            bv = w_v[s, jj, r, pl.ds(D + h * _LANES, _LANES)]
                            plsc.store_scatter(
                                o_v.at[s], [lanes[h], nv], av + scale * bv
                            )

            out_copy(bi, s).start()

        issue_gathers(0, 0)

        @pl.loop(0, (n_blocks - 2) // 2)
        def _(k):
            bi = k * 2
            run_block(bi, 0, True)
            run_block(bi + 1, 1, True)

        run_block(n_blocks - 2, 0, True)
        run_block(n_blocks - 1, 1, False)
        out_copy(n_blocks - 2, 0).wait()
        out_copy(n_blocks - 1, 1).wait()

    return fused


@jax.jit
def kernel(x, Wa, Wb):
    N, T = x.shape
    D = Wa.shape[1]
    V = Wa.shape[0]
    w = jnp.concatenate(
        [Wa, Wb, jnp.zeros((V, 128 - 2 * D), jnp.float32)], axis=1
    )
    xt3 = x.T.astype(jnp.int32).reshape(T, N // 64, 64)
    out_t = _build(N, T, D)(xt3, w)
    return out_t.transpose(2, 0, 1)
